# baseline (speedup 1.0000x reference)
import functools

import jax
import jax.numpy as jnp
import numpy as np
from jax import lax
from jax.experimental import pallas as pl
from jax.experimental.pallas import tpu as pltpu
from jax.experimental.pallas import tpu_sc as plsc

NUM_STEPS = 1000
BETA_START = 0.0001
BETA_END = 0.02

_beta = np.linspace(BETA_START, BETA_END, NUM_STEPS, dtype=np.float32)
_alpha_bar = np.cumprod((1.0 - _beta).astype(np.float64))
_SQRT_AB = np.sqrt(_alpha_bar).astype(np.float32)
_SQRT_1MAB = np.sqrt(1.0 - _alpha_bar).astype(np.float32)

_LANES = 16


def _sc_gather(t, sa_tab, sv_tab):
    b = t.shape[0]
    n_tab = sa_tab.shape[0]
    mesh = plsc.VectorSubcoreMesh(core_axis_name="c", subcore_axis_name="s")

    @functools.partial(
        pl.kernel,
        out_type=(
            jax.ShapeDtypeStruct((b,), jnp.float32),
            jax.ShapeDtypeStruct((b,), jnp.float32),
        ),
        mesh=mesh,
        compiler_params=pltpu.CompilerParams(needs_layout_passes=False),
        scratch_types=[
            pltpu.VMEM((b,), jnp.int32),
            pltpu.VMEM((n_tab,), jnp.float32),
            pltpu.VMEM((n_tab,), jnp.float32),
            pltpu.VMEM((b,), jnp.float32),
            pltpu.VMEM((b,), jnp.float32),
        ],
    )
    def k(t_hbm, sa_hbm, sv_hbm, sa_out, sv_out, idx_v, sa_tab_v, sv_tab_v, sa_v, sv_v):
        @pl.when((lax.axis_index("c") == 0) & (lax.axis_index("s") == 0))
        def _():
            pltpu.sync_copy(t_hbm, idx_v)
            pltpu.sync_copy(sa_hbm, sa_tab_v)
            pltpu.sync_copy(sv_hbm, sv_tab_v)
            for g in range(b // _LANES):
                ti = idx_v[pl.ds(g * _LANES, _LANES)]
                sa_v[pl.ds(g * _LANES, _LANES)] = plsc.load_gather(sa_tab_v, [ti])
                sv_v[pl.ds(g * _LANES, _LANES)] = plsc.load_gather(sv_tab_v, [ti])
            pltpu.sync_copy(sa_v, sa_out)
            pltpu.sync_copy(sv_v, sv_out)

    return k(t, sa_tab, sv_tab)


def _tc_body(sa_ref, sv_ref, x_ref, e_ref, o_ref):
    o_ref[...] = sa_ref[...] * x_ref[...] + sv_ref[...] * e_ref[...]


def _tc_fma(sa, sv, x, e, chunk):
    b, n = x.shape
    grid = (n // chunk,)
    return pl.pallas_call(
        _tc_body,
        grid=grid,
        in_specs=[
            pl.BlockSpec((b, 1), lambda k: (0, 0)),
            pl.BlockSpec((b, 1), lambda k: (0, 0)),
            pl.BlockSpec((b, chunk), lambda k: (0, k)),
            pl.BlockSpec((b, chunk), lambda k: (0, k)),
        ],
        out_specs=pl.BlockSpec((b, chunk), lambda k: (0, k)),
        out_shape=jax.ShapeDtypeStruct((b, n), jnp.float32),
        compiler_params=pltpu.CompilerParams(
            dimension_semantics=("arbitrary",),
        ),
    )(sa, sv, x, e)


def kernel(x0, t, eps):
    b = x0.shape[0]
    n = x0.size // b
    xr = x0.reshape(b, n)
    er = eps.reshape(b, n)
    t32 = t.astype(jnp.int32)
    sa_t, sv_t = _sc_gather(t32, jnp.asarray(_SQRT_AB), jnp.asarray(_SQRT_1MAB))
    out = _tc_fma(sa_t.reshape(b, 1), sv_t.reshape(b, 1), xr, er, chunk=4096)
    return out.reshape(x0.shape)


# speedup vs baseline: 2.2196x; 2.2196x over previous
import functools

import jax
import jax.numpy as jnp
import numpy as np
from jax import lax
from jax.experimental import pallas as pl
from jax.experimental.pallas import tpu as pltpu
from jax.experimental.pallas import tpu_sc as plsc

NUM_STEPS = 1000
BETA_START = 0.0001
BETA_END = 0.02

_beta = np.linspace(BETA_START, BETA_END, NUM_STEPS, dtype=np.float32)
_alpha_bar = np.cumprod((1.0 - _beta).astype(np.float64))
_SQRT_AB = np.sqrt(_alpha_bar).astype(np.float32)
_SQRT_1MAB = np.sqrt(1.0 - _alpha_bar).astype(np.float32)

_LANES = 16


def _sc_gather(t, sa_tab, sv_tab):
    b = t.shape[0]
    n_tab = sa_tab.shape[0]
    mesh = plsc.VectorSubcoreMesh(core_axis_name="c", subcore_axis_name="s")

    @functools.partial(
        pl.kernel,
        out_type=(
            jax.ShapeDtypeStruct((b,), jnp.float32),
            jax.ShapeDtypeStruct((b,), jnp.float32),
        ),
        mesh=mesh,
        compiler_params=pltpu.CompilerParams(needs_layout_passes=False),
        scratch_types=[
            pltpu.VMEM((b,), jnp.int32),
            pltpu.VMEM((n_tab,), jnp.float32),
            pltpu.VMEM((n_tab,), jnp.float32),
            pltpu.VMEM((b,), jnp.float32),
            pltpu.VMEM((b,), jnp.float32),
        ],
    )
    def k(t_hbm, sa_hbm, sv_hbm, sa_out, sv_out, idx_v, sa_tab_v, sv_tab_v, sa_v, sv_v):
        @pl.when((lax.axis_index("c") == 0) & (lax.axis_index("s") == 0))
        def _():
            pltpu.sync_copy(t_hbm, idx_v)
            pltpu.sync_copy(sa_hbm, sa_tab_v)
            pltpu.sync_copy(sv_hbm, sv_tab_v)
            for g in range(b // _LANES):
                ti = idx_v[pl.ds(g * _LANES, _LANES)]
                sa_v[pl.ds(g * _LANES, _LANES)] = plsc.load_gather(sa_tab_v, [ti])
                sv_v[pl.ds(g * _LANES, _LANES)] = plsc.load_gather(sv_tab_v, [ti])
            pltpu.sync_copy(sa_v, sa_out)
            pltpu.sync_copy(sv_v, sv_out)

    return k(t, sa_tab, sv_tab)


def _tc_body(sa_ref, sv_ref, x_ref, e_ref, o_ref):
    b = pl.program_id(0)
    a = sa_ref[b]
    v = sv_ref[b]
    o_ref[...] = a * x_ref[...] + v * e_ref[...]


def _tc_fma(sa, sv, x, e, rows):
    b, c, h, w = x.shape
    grid = (b, c, h // rows)
    blk = pl.BlockSpec((1, 1, rows, w), lambda i, j, k: (i, j, k, 0))
    return pl.pallas_call(
        _tc_body,
        grid=grid,
        in_specs=[
            pl.BlockSpec(memory_space=pltpu.SMEM),
            pl.BlockSpec(memory_space=pltpu.SMEM),
            blk,
            blk,
        ],
        out_specs=blk,
        out_shape=jax.ShapeDtypeStruct(x.shape, jnp.float32),
        compiler_params=pltpu.CompilerParams(
            dimension_semantics=("parallel", "parallel", "parallel"),
        ),
    )(sa, sv, x, e)


def kernel(x0, t, eps):
    t32 = t.astype(jnp.int32)
    sa_t, sv_t = _sc_gather(t32, jnp.asarray(_SQRT_AB), jnp.asarray(_SQRT_1MAB))
    return _tc_fma(sa_t, sv_t, x0, eps, rows=256)


# speedup vs baseline: 3.0106x; 1.3564x over previous
import functools

import jax
import jax.numpy as jnp
import numpy as np
from jax import lax
from jax.experimental import pallas as pl
from jax.experimental.pallas import tpu as pltpu
from jax.experimental.pallas import tpu_sc as plsc

NUM_STEPS = 1000
BETA_START = 0.0001
BETA_END = 0.02

_beta = np.linspace(BETA_START, BETA_END, NUM_STEPS, dtype=np.float32)
_alpha_bar = np.cumprod((1.0 - _beta).astype(np.float64))
_SQRT_AB = np.sqrt(_alpha_bar).astype(np.float32)
_SQRT_1MAB = np.sqrt(1.0 - _alpha_bar).astype(np.float32)

_LANES = 16


def _sc_gather(t, sa_tab, sv_tab):
    b = t.shape[0]
    n_tab = sa_tab.shape[0]
    mesh = plsc.VectorSubcoreMesh(core_axis_name="c", subcore_axis_name="s")

    @functools.partial(
        pl.kernel,
        out_type=(
            jax.ShapeDtypeStruct((b,), jnp.float32),
            jax.ShapeDtypeStruct((b,), jnp.float32),
        ),
        mesh=mesh,
        compiler_params=pltpu.CompilerParams(needs_layout_passes=False),
        scratch_types=[
            pltpu.VMEM((b,), jnp.int32),
            pltpu.VMEM((n_tab,), jnp.float32),
            pltpu.VMEM((n_tab,), jnp.float32),
            pltpu.VMEM((b,), jnp.float32),
            pltpu.VMEM((b,), jnp.float32),
        ],
    )
    def k(t_hbm, sa_hbm, sv_hbm, sa_out, sv_out, idx_v, sa_tab_v, sv_tab_v, sa_v, sv_v):
        @pl.when((lax.axis_index("c") == 0) & (lax.axis_index("s") == 0))
        def _():
            pltpu.sync_copy(t_hbm, idx_v)
            pltpu.sync_copy(sa_hbm, sa_tab_v)
            pltpu.sync_copy(sv_hbm, sv_tab_v)
            for g in range(b // _LANES):
                ti = idx_v[pl.ds(g * _LANES, _LANES)]
                sa_v[pl.ds(g * _LANES, _LANES)] = plsc.load_gather(sa_tab_v, [ti])
                sv_v[pl.ds(g * _LANES, _LANES)] = plsc.load_gather(sv_tab_v, [ti])
            pltpu.sync_copy(sa_v, sa_out)
            pltpu.sync_copy(sv_v, sv_out)

    return k(t, sa_tab, sv_tab)


def _tc_body(sa_ref, sv_ref, x_ref, e_ref, o_ref):
    b = pl.program_id(0)
    a = sa_ref[b]
    v = sv_ref[b]
    o_ref[...] = a * x_ref[...] + v * e_ref[...]


def _tc_fma(sa, sv, x, e, rows):
    b, c, h, w = x.shape
    grid = (b, c, h // rows)
    blk = pl.BlockSpec((1, 1, rows, w), lambda i, j, k: (i, j, k, 0))
    return pl.pallas_call(
        _tc_body,
        grid=grid,
        in_specs=[
            pl.BlockSpec(memory_space=pltpu.SMEM),
            pl.BlockSpec(memory_space=pltpu.SMEM),
            blk,
            blk,
        ],
        out_specs=blk,
        out_shape=jax.ShapeDtypeStruct(x.shape, jnp.float32),
        compiler_params=pltpu.CompilerParams(
            dimension_semantics=("parallel", "parallel", "parallel"),
        ),
    )(sa, sv, x, e)


def kernel(x0, t, eps):
    t32 = t.astype(jnp.int32)
    sa_t, sv_t = _sc_gather(t32, jnp.asarray(_SQRT_AB), jnp.asarray(_SQRT_1MAB))
    return _tc_fma(sa_t, sv_t, x0, eps, rows=512)


# speedup vs baseline: 3.5613x; 1.1829x over previous
import functools

import jax
import jax.numpy as jnp
import numpy as np
from jax import lax
from jax.experimental import pallas as pl
from jax.experimental.pallas import tpu as pltpu
from jax.experimental.pallas import tpu_sc as plsc

NUM_STEPS = 1000
BETA_START = 0.0001
BETA_END = 0.02

_beta = np.linspace(BETA_START, BETA_END, NUM_STEPS, dtype=np.float32)
_alpha_bar = np.cumprod((1.0 - _beta).astype(np.float64))
_SQRT_AB = np.sqrt(_alpha_bar).astype(np.float32)
_SQRT_1MAB = np.sqrt(1.0 - _alpha_bar).astype(np.float32)

_LANES = 16


def _sc_gather(t, sa_tab, sv_tab):
    b = t.shape[0]
    n_tab = sa_tab.shape[0]
    mesh = plsc.VectorSubcoreMesh(core_axis_name="c", subcore_axis_name="s")

    @functools.partial(
        pl.kernel,
        out_type=(
            jax.ShapeDtypeStruct((b,), jnp.float32),
            jax.ShapeDtypeStruct((b,), jnp.float32),
        ),
        mesh=mesh,
        compiler_params=pltpu.CompilerParams(needs_layout_passes=False),
        scratch_types=[
            pltpu.VMEM((b,), jnp.int32),
            pltpu.VMEM((n_tab,), jnp.float32),
            pltpu.VMEM((n_tab,), jnp.float32),
            pltpu.VMEM((b,), jnp.float32),
            pltpu.VMEM((b,), jnp.float32),
        ],
    )
    def k(t_hbm, sa_hbm, sv_hbm, sa_out, sv_out, idx_v, sa_tab_v, sv_tab_v, sa_v, sv_v):
        @pl.when((lax.axis_index("c") == 0) & (lax.axis_index("s") == 0))
        def _():
            pltpu.sync_copy(t_hbm, idx_v)
            pltpu.sync_copy(sa_hbm, sa_tab_v)
            pltpu.sync_copy(sv_hbm, sv_tab_v)
            for g in range(b // _LANES):
                ti = idx_v[pl.ds(g * _LANES, _LANES)]
                sa_v[pl.ds(g * _LANES, _LANES)] = plsc.load_gather(sa_tab_v, [ti])
                sv_v[pl.ds(g * _LANES, _LANES)] = plsc.load_gather(sv_tab_v, [ti])
            pltpu.sync_copy(sa_v, sa_out)
            pltpu.sync_copy(sv_v, sv_out)

    return k(t, sa_tab, sv_tab)


def _tc_body(sa_ref, sv_ref, x_ref, e_ref, o_ref):
    b = pl.program_id(0)
    a = sa_ref[b]
    v = sv_ref[b]
    o_ref[...] = a * x_ref[...] + v * e_ref[...]


def _tc_fma(sa, sv, x, e):
    b, c, h, w = x.shape
    grid = (b,)
    blk = pl.BlockSpec((1, c, h, w), lambda i: (i, 0, 0, 0))
    return pl.pallas_call(
        _tc_body,
        grid=grid,
        in_specs=[
            pl.BlockSpec(memory_space=pltpu.SMEM),
            pl.BlockSpec(memory_space=pltpu.SMEM),
            blk,
            blk,
        ],
        out_specs=blk,
        out_shape=jax.ShapeDtypeStruct(x.shape, jnp.float32),
        compiler_params=pltpu.CompilerParams(
            dimension_semantics=("parallel",),
        ),
    )(sa, sv, x, e)


def kernel(x0, t, eps):
    t32 = t.astype(jnp.int32)
    sa_t, sv_t = _sc_gather(t32, jnp.asarray(_SQRT_AB), jnp.asarray(_SQRT_1MAB))
    return _tc_fma(sa_t, sv_t, x0, eps)
